# baseline (speedup 1.0000x reference)
import jax
import jax.numpy as jnp
from jax import lax
from jax.experimental import pallas as pl
from jax.experimental.pallas import tpu as pltpu
from jax.experimental.pallas import tpu_sc as plsc

F32 = jnp.float32
N = 10000
E = 320000
G = 64
NC, NS = 2, 16
NW = NC * NS
EPT = E // NW
CH = 80
NCHUNK = EPT // CH
IDXROWS = E // CH
BE = 2560
BN = 2000


def _leaky(v):
    return jnp.where(v >= 0, v, 0.01 * v)


_PREC = lax.Precision.HIGHEST


def _dot(a, b):
    return jnp.dot(a, b, preferred_element_type=F32)


def _hdot(a, b):
    return jnp.dot(a, b, preferred_element_type=F32, precision=_PREC)


def _enc_edge_fused(eat, We, be):
    pre = lax.dot_general(eat[...], We[...], (((0,), (0,)), ((), ())),
                          preferred_element_type=F32, precision=_PREC)
    return _leaky(pre + be[...])


def _enc_node_body(xb, Wn, bn, ub, Wg, bg, h_out, g_out):
    h_out[...] = _leaky(_hdot(xb[...], Wn[...]) + bn[...])
    g_out[...] = _leaky(_hdot(ub[...], Wg[...]) + bg[...])


def _tables_body(hb, bb, gb, Ew1a, Ew1g, Eb1, Ew1b, Nw1aa, Nb1a, trow, tb):
    onehot = (bb[...] == lax.broadcasted_iota(jnp.int32, (BN, G), 1)).astype(F32)
    gterm = _hdot(onehot, _hdot(gb[...], Ew1g[...]))
    ta = _hdot(hb[...], Ew1a[...]) + gterm + Eb1[...]
    tcm = _hdot(hb[...], Nw1aa[...]) + Nb1a[...]
    trow[...] = jnp.concatenate([ta, tcm], axis=1)
    tb[...] = _hdot(hb[...], Ew1b[...])


def _edge1_body(drow, eat, We, be, Ew1c, Ew2, Eb2, Nw1ab, Nw1b, Nb1b,
                e_new, m_out):
    e0 = _enc_edge_fused(eat, We, be)
    pre = drow[:, :64] + _hdot(e0, Ew1c[...])
    hid = _leaky(pre)
    en = _hdot(hid, Ew2[...]) + Eb2[...]
    e_new[...] = en
    m1 = _leaky(drow[:, 64:] + _hdot(en, Nw1ab[...]))
    m_out[...] = _hdot(m1, Nw1b[...]) + Nb1b[...]


def _edge2_body(drow, eb, Ew1c, Ew2, Eb2, Nw1ab, Nw1b, Nb1b, DwT, Db,
                m_out, eoT_out):
    pre = drow[:, :64] + _hdot(eb[...], Ew1c[...])
    hid = _leaky(pre)
    en = _hdot(hid, Ew2[...]) + Eb2[...]
    m1 = _leaky(drow[:, 64:] + _hdot(en, Nw1ab[...]))
    m_out[...] = _hdot(m1, Nw1b[...]) + Nb1b[...]
    eoT_out[...] = lax.dot_general(DwT[...], en, (((1,), (1,)), ((), ())),
                                   preferred_element_type=F32,
                                   precision=_PREC) + Db[...]


def _node_body(hb, aggp, cntp, bc, br, gb,
               Nw2ah, Nw2aa, Nw2ag, Nb2a, Nw2b, Nb2b,
               Gw1g, Gw1n, Gb1, Gw2, Gb2, Vw1, Vb1, Vw2, Vb2,
               h_new, g_new, value):
    cnt = cntp[0, :, 0:1] + cntp[1, :, 0:1]
    agg = (aggp[0] + aggp[1]) / jnp.maximum(cnt, 1.0)
    onehot = (bc[...] == lax.broadcasted_iota(jnp.int32, (N, G), 1)).astype(F32)
    onehotT = (br[...] == lax.broadcasted_iota(jnp.int32, (G, N), 0)).astype(F32)
    gn = _dot(onehot, _hdot(gb[...], Nw2ag[...])) + Nb2a[...]
    hid = _leaky(_dot(hb[...], Nw2ah[...]) + _dot(agg, Nw2aa[...]) + gn)
    hn = _dot(hid, Nw2b[...]) + Nb2b[...]
    h_new[...] = hn
    nsum = _hdot(onehotT, hn)
    ncnt = _hdot(onehotT, jnp.ones((N, 1), F32))
    nmean = nsum / jnp.maximum(ncnt, 1.0)
    ghid = _leaky(_hdot(gb[...], Gw1g[...]) + _hdot(nmean, Gw1n[...]) + Gb1[...])
    gnew = _hdot(ghid, Gw2[...]) + Gb2[...]
    g_new[...] = gnew
    value[...] = _hdot(_leaky(_hdot(gnew, Vw1[...]) + Vb1[...]), Vw2[...]) + Vb2[...]


def _full(shape):
    return pl.BlockSpec(shape, lambda i: (0, 0))


def _enc_node(x, Wn, bn, u, Wg, bg):
    return pl.pallas_call(
        _enc_node_body,
        grid=(N // BN,),
        in_specs=[pl.BlockSpec((BN, 128), lambda i: (i, 0)),
                  _full((128, 64)), _full((1, 64)),
                  _full((G, 32)), _full((32, 32)), _full((1, 32))],
        out_specs=[pl.BlockSpec((BN, 64), lambda i: (i, 0)),
                   pl.BlockSpec((G, 32), lambda i: (0, 0))],
        out_shape=[jax.ShapeDtypeStruct((N, 64), F32),
                   jax.ShapeDtypeStruct((G, 32), F32)],
    )(x, Wn, bn, u, Wg, bg)


def _tables(h, batch_c, g, Ew1a, Ew1g, Eb1, Ew1b, Nw1aa, Nb1a):
    return pl.pallas_call(
        _tables_body,
        grid=(N // BN,),
        in_specs=[pl.BlockSpec((BN, 64), lambda i: (i, 0)),
                  pl.BlockSpec((BN, 1), lambda i: (i, 0)),
                  _full((G, 32)),
                  _full((64, 64)), _full((32, 64)), _full((1, 64)),
                  _full((64, 64)), _full((64, 64)), _full((1, 64))],
        out_specs=[pl.BlockSpec((BN, 128), lambda i: (i, 0)),
                   pl.BlockSpec((BN, 64), lambda i: (i, 0))],
        out_shape=[jax.ShapeDtypeStruct((N, 128), F32),
                   jax.ShapeDtypeStruct((N, 64), F32)],
    )(h, batch_c, g, Ew1a, Ew1g, Eb1, Ew1b, Nw1aa, Nb1a)


def _edge_stage1(drow, eat, We, be, Ew1c, Ew2, Eb2, Nw1ab, Nw1b, Nb1b):
    return pl.pallas_call(
        _edge1_body,
        grid=(E // BE,),
        in_specs=[pl.BlockSpec((BE, 128), lambda i: (i, 0)),
                  pl.BlockSpec((16, BE), lambda i: (0, i)),
                  _full((16, 32)), _full((1, 32)),
                  _full((32, 64)), _full((64, 32)), _full((1, 32)),
                  _full((32, 64)), _full((64, 64)), _full((1, 64))],
        out_specs=[pl.BlockSpec((BE, 32), lambda i: (i, 0)),
                   pl.BlockSpec((BE, 64), lambda i: (i, 0))],
        out_shape=[jax.ShapeDtypeStruct((E, 32), F32),
                   jax.ShapeDtypeStruct((E, 64), F32)],
    )(drow, eat, We, be, Ew1c, Ew2, Eb2, Nw1ab, Nw1b, Nb1b)


def _edge_stage2(drow, e, Ew1c, Ew2, Eb2, Nw1ab, Nw1b, Nb1b, DwT, Db):
    return pl.pallas_call(
        _edge2_body,
        grid=(E // BE,),
        in_specs=[pl.BlockSpec((BE, 128), lambda i: (i, 0)),
                  pl.BlockSpec((BE, 32), lambda i: (i, 0)),
                  _full((32, 64)), _full((64, 32)), _full((1, 32)),
                  _full((32, 64)), _full((64, 64)), _full((1, 64)),
                  _full((1, 32)), _full((1, 1))],
        out_specs=[pl.BlockSpec((BE, 64), lambda i: (i, 0)),
                   pl.BlockSpec((1, BE), lambda i: (0, i))],
        out_shape=[jax.ShapeDtypeStruct((E, 64), F32),
                   jax.ShapeDtypeStruct((1, E), F32)],
    )(drow, e, Ew1c, Ew2, Eb2, Nw1ab, Nw1b, Nb1b, DwT, Db)


def _node_stage(h, aggp, cntp, batch_c, batch_r, g, *weights):
    return pl.pallas_call(
        _node_body,
        out_shape=[jax.ShapeDtypeStruct((N, 64), F32),
                   jax.ShapeDtypeStruct((G, 32), F32),
                   jax.ShapeDtypeStruct((G, 1), F32)],
        compiler_params=pltpu.CompilerParams(vmem_limit_bytes=100 * 1024 * 1024),
    )(h, aggp, cntp, batch_c, batch_r, g, *weights)


_MESH = plsc.VectorSubcoreMesh(core_axis_name="c", subcore_axis_name="s",
                               num_cores=NC, num_subcores=NS)
_SC_PARAMS = pltpu.CompilerParams(use_tc_tiling_on_sc=False)


def _gather_kernel(trow, tb, rowm, colm, drow,
                   rowv, colv, grbuf, gcbuf, gsem, ssem):
    cid = lax.axis_index("c")
    sid = lax.axis_index("s")
    wid = sid * NC + cid
    ebase = wid * EPT
    pltpu.sync_copy(rowm.at[wid], rowv)
    pltpu.sync_copy(colm.at[wid], colv)
    pltpu.async_copy(trow.at[rowv.at[0]], grbuf.at[0], gsem)
    pltpu.async_copy(tb.at[colv.at[0]], gcbuf.at[0], gsem)

    def body(ci, carry):
        slot = lax.rem(ci, 2)
        other = 1 - slot

        @pl.when(ci >= 1)
        def _():
            pltpu.make_async_copy(grbuf.at[other], drow.at[pl.ds(0, CH)], ssem).wait()

        @pl.when(ci + 1 < NCHUNK)
        def _():
            pltpu.async_copy(trow.at[rowv.at[ci + 1]], grbuf.at[other], gsem)
            pltpu.async_copy(tb.at[colv.at[ci + 1]], gcbuf.at[other], gsem)

        pltpu.make_async_copy(trow.at[rowv.at[ci]], grbuf.at[slot], gsem).wait()
        pltpu.make_async_copy(tb.at[colv.at[ci]], gcbuf.at[slot], gsem).wait()

        def add_body(r, c):
            for j in range(4):
                sl = pl.ds(j * 16, 16)
                grbuf[slot, r, sl] = grbuf[slot, r, sl] + gcbuf[slot, r, sl]
            return c

        lax.fori_loop(0, CH, add_body, 0, unroll=8)
        off = ebase + ci * CH
        pltpu.async_copy(grbuf.at[slot], drow.at[pl.ds(off, CH)], ssem)
        return carry

    lax.fori_loop(0, NCHUNK, body, 0)
    pltpu.make_async_copy(grbuf.at[0], drow.at[pl.ds(0, CH)], ssem).wait()


def _gather(trow, tb, rowm, colm):
    kern = pl.kernel(
        _gather_kernel,
        out_type=jax.ShapeDtypeStruct((E, 128), F32),
        mesh=_MESH,
        scratch_types=[pltpu.VMEM((NCHUNK, CH), jnp.int32),
                       pltpu.VMEM((NCHUNK, CH), jnp.int32),
                       pltpu.VMEM((2, CH, 128), F32),
                       pltpu.VMEM((2, CH, 64), F32),
                       pltpu.SemaphoreType.DMA,
                       pltpu.SemaphoreType.DMA],
        compiler_params=_SC_PARAMS,
    )
    return kern(trow, tb, rowm, colm)


def _scatter_kernel(m, colm, zeros, aggp, colv, mbuf, aggsh, sem):
    cid = lax.axis_index("c")
    sid = lax.axis_index("s")
    wid = sid * NC + cid

    @pl.when(sid < 10)
    def _():
        pltpu.sync_copy(zeros, aggsh.at[pl.ds(sid * 1000, 1000)])

    pltpu.sync_copy(colm.at[wid], colv)
    plsc.subcore_barrier()
    pltpu.async_copy(m.at[pl.ds(wid * EPT, CH)], mbuf.at[0], sem)

    def body(ci, carry):
        slot = lax.rem(ci, 2)

        @pl.when(ci + 1 < NCHUNK)
        def _():
            off = wid * EPT + (ci + 1) * CH
            pltpu.async_copy(m.at[pl.ds(off, CH)], mbuf.at[1 - slot], sem)

        pltpu.make_async_copy(m.at[pl.ds(0, CH)], mbuf.at[slot], sem).wait()
        pltpu.sync_copy(mbuf.at[slot], aggsh.at[colv.at[ci]], add=True)
        return carry

    lax.fori_loop(0, NCHUNK, body, 0)
    plsc.subcore_barrier()

    @pl.when(sid < 10)
    def _():
        pltpu.sync_copy(aggsh.at[pl.ds(sid * 1000, 1000)],
                        aggp.at[cid, pl.ds(sid * 1000, 1000)])


def _scatter(m, colm, zeros):
    kern = pl.kernel(
        _scatter_kernel,
        out_type=jax.ShapeDtypeStruct((NC, N, 64), F32),
        mesh=_MESH,
        scratch_types=[pltpu.VMEM((NCHUNK, CH), jnp.int32),
                       pltpu.VMEM((2, CH, 64), F32),
                       pltpu.VMEM_SHARED((N, 64), F32),
                       pltpu.SemaphoreType.DMA],
        compiler_params=_SC_PARAMS,
    )
    return kern(m, colm, zeros)


def _cnt_kernel(colm, zeros16, ones16, cntp, colv, obuf, cntsh):
    cid = lax.axis_index("c")
    sid = lax.axis_index("s")
    wid = sid * NC + cid

    @pl.when(sid < 10)
    def _():
        pltpu.sync_copy(zeros16, cntsh.at[pl.ds(sid * 1000, 1000)])

    pltpu.sync_copy(ones16, obuf)
    pltpu.sync_copy(colm.at[wid], colv)
    plsc.subcore_barrier()

    def body(ci, carry):
        pltpu.sync_copy(obuf, cntsh.at[colv.at[ci]], add=True)
        return carry

    lax.fori_loop(0, NCHUNK, body, 0)
    plsc.subcore_barrier()

    @pl.when(sid < 10)
    def _():
        pltpu.sync_copy(cntsh.at[pl.ds(sid * 1000, 1000)],
                        cntp.at[cid, pl.ds(sid * 1000, 1000)])


def _cnt(colm, zeros16, ones16):
    kern = pl.kernel(
        _cnt_kernel,
        out_type=jax.ShapeDtypeStruct((NC, N, 16), F32),
        mesh=_MESH,
        scratch_types=[pltpu.VMEM((NCHUNK, CH), jnp.int32),
                       pltpu.VMEM((CH, 16), F32),
                       pltpu.VMEM_SHARED((N, 16), F32)],
        compiler_params=_SC_PARAMS,
    )
    return kern(colm, zeros16, ones16)


def kernel(x, edge_index, edge_attr, u, batch, params):
    p = params
    row = edge_index[0]
    col = edge_index[1]
    rowm = row.reshape(NW, NCHUNK, CH)
    colm = col.reshape(NW, NCHUNK, CH)
    batch_c = batch.reshape(N, 1)
    batch_r = batch.reshape(1, N)
    zeros64 = jnp.zeros((1000, 64), F32)
    zeros16 = jnp.zeros((1000, 16), F32)
    ones16 = jnp.ones((CH, 16), F32)

    Ew1 = p['Ew1']
    Nw1a = p['Nw1a']
    Nw2a = p['Nw2a']
    Gw1 = p['Gw1']

    def r2(b):
        return b.reshape(1, -1)

    eat = edge_attr.T
    h, g = _enc_node(x, p['Wn'], r2(p['bn']), u, p['Wg'], r2(p['bg']))
    cntp = _cnt(colm, zeros16, ones16)

    node_w = (Nw2a[0:64], Nw2a[64:128], Nw2a[128:160], r2(p['Nb2a']),
              p['Nw2b'], r2(p['Nb2b']),
              Gw1[0:32], Gw1[32:96], r2(p['Gb1']), p['Gw2'], r2(p['Gb2']),
              p['Vw1'], r2(p['Vb1']), p['Vw2'], r2(p['Vb2']))

    def tables_gather(h, g):
        trow, tb = _tables(h, batch_c, g,
                           Ew1[0:64], Ew1[160:192], r2(p['Eb1']),
                           Ew1[64:128], Nw1a[0:64], r2(p['Nb1a']))
        return _gather(trow, tb, rowm, colm)

    edge_w = (Ew1[128:160], p['Ew2'], r2(p['Eb2']),
              Nw1a[64:96], p['Nw1b'], r2(p['Nb1b']))

    drow = tables_gather(h, g)
    e, m = _edge_stage1(drow, eat, p['We'], r2(p['be']), *edge_w)
    aggp = _scatter(m, colm, zeros64)
    h, g, _ = _node_stage(h, aggp, cntp, batch_c, batch_r, g, *node_w)

    drow = tables_gather(h, g)
    m, eoT = _edge_stage2(drow, e, *edge_w, p['Dw'].reshape(1, 32), r2(p['Db']))
    aggp = _scatter(m, colm, zeros64)
    h, g, value = _node_stage(h, aggp, cntp, batch_c, batch_r, g, *node_w)

    return eoT.reshape(E, 1), value


# speedup vs baseline: 1.9641x; 1.9641x over previous
import jax
import jax.numpy as jnp
from jax import lax
from jax.experimental import pallas as pl
from jax.experimental.pallas import tpu as pltpu
from jax.experimental.pallas import tpu_sc as plsc

F32 = jnp.float32
N = 10000
E = 320000
G = 64
NC, NS = 2, 16
NW = NC * NS
EPT = E // NW
CH = 80
NCHUNK = EPT // CH
BE = 2560
BN = 2000

_PREC = lax.Precision.HIGHEST


def _leaky(v):
    return jnp.where(v >= 0, v, 0.01 * v)


def _dot(a, b):
    return jnp.dot(a, b, preferred_element_type=F32)


def _hdot(a, b):
    return jnp.dot(a, b, preferred_element_type=F32, precision=_PREC)


def _enc_node_body(xb, Wn, bn, ub, Wg, bg, h_out, g_out):
    h_out[...] = _leaky(_dot(xb[...], Wn[...]) + bn[...])
    g_out[...] = _leaky(_dot(ub[...], Wg[...]) + bg[...])


def _tables_body(hb, bb, gb, Ew1a, Ew1g, Eb1, Ew1b, Nw1aa, Nb1a, trow, tb):
    onehot = (bb[...] == lax.broadcasted_iota(jnp.int32, (BN, G), 1)).astype(F32)
    gterm = _dot(onehot, _hdot(gb[...], Ew1g[...]))
    ta = _dot(hb[...], Ew1a[...]) + gterm + Eb1[...]
    tcm = _dot(hb[...], Nw1aa[...]) + Nb1a[...]
    trow[...] = jnp.concatenate([ta, tcm], axis=1)
    tb[...] = _dot(hb[...], Ew1b[...])


def _enc_edge_fused(eat, We, be):
    pre = lax.dot_general(eat[...], We[...], (((0,), (0,)), ((), ())),
                          preferred_element_type=F32)
    return _leaky(pre + be[...])


def _edge1_body(drow, db, eat, We, be, Ew1c, Ew2, Eb2, Nw1ab, Nw1b, Nb1b,
                e_new, m_out):
    e0 = _enc_edge_fused(eat, We, be)
    pre = drow[:, :64] + db[...] + _dot(e0, Ew1c[...])
    hid = _leaky(pre)
    en = _dot(hid, Ew2[...]) + Eb2[...]
    e_new[...] = en
    m1 = _leaky(drow[:, 64:] + _dot(en, Nw1ab[...]))
    m_out[...] = _dot(m1, Nw1b[...]) + Nb1b[...]


def _edge2_body(drow, db, eb, Ew1c, Ew2, Eb2, Nw1ab, Nw1b, Nb1b, DwT, Db,
                m_out, eoT_out):
    pre = drow[:, :64] + db[...] + _dot(eb[...], Ew1c[...])
    hid = _leaky(pre)
    en = _dot(hid, Ew2[...]) + Eb2[...]
    m1 = _leaky(drow[:, 64:] + _dot(en, Nw1ab[...]))
    m_out[...] = _dot(m1, Nw1b[...]) + Nb1b[...]
    eoT_out[...] = lax.dot_general(DwT[...], en, (((1,), (1,)), ((), ())),
                                   preferred_element_type=F32) + Db[...]


def _node_body(hb, aggp, cntp, bc, br, gb,
               Nw2ah, Nw2aa, Nw2ag, Nb2a, Nw2b, Nb2b,
               Gw1g, Gw1n, Gb1, Gw2, Gb2, Vw1, Vb1, Vw2, Vb2,
               h_new, g_new, value):
    cnt = cntp[0, :, 0:1] + cntp[1, :, 0:1]
    agg = (aggp[0] + aggp[1]) / jnp.maximum(cnt, 1.0)
    onehot = (bc[...] == lax.broadcasted_iota(jnp.int32, (N, G), 1)).astype(F32)
    onehotT = (br[...] == lax.broadcasted_iota(jnp.int32, (G, N), 0)).astype(F32)
    gn = _dot(onehot, _hdot(gb[...], Nw2ag[...])) + Nb2a[...]
    hid = _leaky(_dot(hb[...], Nw2ah[...]) + _dot(agg, Nw2aa[...]) + gn)
    hn = _dot(hid, Nw2b[...]) + Nb2b[...]
    h_new[...] = hn
    nsum = _hdot(onehotT, hn)
    ncnt = _hdot(onehotT, jnp.ones((N, 1), F32))
    nmean = nsum / jnp.maximum(ncnt, 1.0)
    ghid = _leaky(_hdot(gb[...], Gw1g[...]) + _hdot(nmean, Gw1n[...]) + Gb1[...])
    gnew = _hdot(ghid, Gw2[...]) + Gb2[...]
    g_new[...] = gnew
    value[...] = _hdot(_leaky(_hdot(gnew, Vw1[...]) + Vb1[...]), Vw2[...]) + Vb2[...]


def _full(shape):
    return pl.BlockSpec(shape, lambda i: (0, 0))


def _enc_node(x, Wn, bn, u, Wg, bg):
    return pl.pallas_call(
        _enc_node_body,
        grid=(N // BN,),
        in_specs=[pl.BlockSpec((BN, 128), lambda i: (i, 0)),
                  _full((128, 64)), _full((1, 64)),
                  _full((G, 32)), _full((32, 32)), _full((1, 32))],
        out_specs=[pl.BlockSpec((BN, 64), lambda i: (i, 0)),
                   pl.BlockSpec((G, 32), lambda i: (0, 0))],
        out_shape=[jax.ShapeDtypeStruct((N, 64), F32),
                   jax.ShapeDtypeStruct((G, 32), F32)],
    )(x, Wn, bn, u, Wg, bg)


def _tables(h, batch_c, g, Ew1a, Ew1g, Eb1, Ew1b, Nw1aa, Nb1a):
    return pl.pallas_call(
        _tables_body,
        grid=(N // BN,),
        in_specs=[pl.BlockSpec((BN, 64), lambda i: (i, 0)),
                  pl.BlockSpec((BN, 1), lambda i: (i, 0)),
                  _full((G, 32)),
                  _full((64, 64)), _full((32, 64)), _full((1, 64)),
                  _full((64, 64)), _full((64, 64)), _full((1, 64))],
        out_specs=[pl.BlockSpec((BN, 128), lambda i: (i, 0)),
                   pl.BlockSpec((BN, 64), lambda i: (i, 0))],
        out_shape=[jax.ShapeDtypeStruct((N, 128), F32),
                   jax.ShapeDtypeStruct((N, 64), F32)],
    )(h, batch_c, g, Ew1a, Ew1g, Eb1, Ew1b, Nw1aa, Nb1a)


def _edge_stage1(drow, db, eat, We, be, Ew1c, Ew2, Eb2, Nw1ab, Nw1b, Nb1b):
    return pl.pallas_call(
        _edge1_body,
        grid=(E // BE,),
        in_specs=[pl.BlockSpec((BE, 128), lambda i: (i, 0)),
                  pl.BlockSpec((BE, 64), lambda i: (i, 0)),
                  pl.BlockSpec((16, BE), lambda i: (0, i)),
                  _full((16, 32)), _full((1, 32)),
                  _full((32, 64)), _full((64, 32)), _full((1, 32)),
                  _full((32, 64)), _full((64, 64)), _full((1, 64))],
        out_specs=[pl.BlockSpec((BE, 32), lambda i: (i, 0)),
                   pl.BlockSpec((BE, 64), lambda i: (i, 0))],
        out_shape=[jax.ShapeDtypeStruct((E, 32), F32),
                   jax.ShapeDtypeStruct((E, 64), F32)],
    )(drow, db, eat, We, be, Ew1c, Ew2, Eb2, Nw1ab, Nw1b, Nb1b)


def _edge_stage2(drow, db, e, Ew1c, Ew2, Eb2, Nw1ab, Nw1b, Nb1b, DwT, Db):
    return pl.pallas_call(
        _edge2_body,
        grid=(E // BE,),
        in_specs=[pl.BlockSpec((BE, 128), lambda i: (i, 0)),
                  pl.BlockSpec((BE, 64), lambda i: (i, 0)),
                  pl.BlockSpec((BE, 32), lambda i: (i, 0)),
                  _full((32, 64)), _full((64, 32)), _full((1, 32)),
                  _full((32, 64)), _full((64, 64)), _full((1, 64)),
                  _full((1, 32)), _full((1, 1))],
        out_specs=[pl.BlockSpec((BE, 64), lambda i: (i, 0)),
                   pl.BlockSpec((1, BE), lambda i: (0, i))],
        out_shape=[jax.ShapeDtypeStruct((E, 64), F32),
                   jax.ShapeDtypeStruct((1, E), F32)],
    )(drow, db, e, Ew1c, Ew2, Eb2, Nw1ab, Nw1b, Nb1b, DwT, Db)


def _node_stage(h, aggp, cntp, batch_c, batch_r, g, *weights):
    return pl.pallas_call(
        _node_body,
        out_shape=[jax.ShapeDtypeStruct((N, 64), F32),
                   jax.ShapeDtypeStruct((G, 32), F32),
                   jax.ShapeDtypeStruct((G, 1), F32)],
        compiler_params=pltpu.CompilerParams(vmem_limit_bytes=100 * 1024 * 1024),
    )(h, aggp, cntp, batch_c, batch_r, g, *weights)


_MESH = plsc.VectorSubcoreMesh(core_axis_name="c", subcore_axis_name="s",
                               num_cores=NC, num_subcores=NS)
_SC_PARAMS = pltpu.CompilerParams(use_tc_tiling_on_sc=False)


def _gather_kernel(trow, tb, rowm, colm, drow, db,
                   rowv, colv, grbuf, gcbuf, gsem, ssem):
    cid = lax.axis_index("c")
    sid = lax.axis_index("s")
    wid = sid * NC + cid
    ebase = wid * EPT
    pltpu.sync_copy(rowm.at[wid], rowv)
    pltpu.sync_copy(colm.at[wid], colv)
    pltpu.async_copy(trow.at[rowv.at[0]], grbuf.at[0], gsem)
    pltpu.async_copy(tb.at[colv.at[0]], gcbuf.at[0], gsem)

    def body(ci, carry):
        slot = lax.rem(ci, 2)
        other = 1 - slot

        @pl.when(ci >= 1)
        def _():
            pltpu.make_async_copy(grbuf.at[other], drow.at[pl.ds(0, CH)], ssem).wait()
            pltpu.make_async_copy(gcbuf.at[other], db.at[pl.ds(0, CH)], ssem).wait()

        @pl.when(ci + 1 < NCHUNK)
        def _():
            pltpu.async_copy(trow.at[rowv.at[ci + 1]], grbuf.at[other], gsem)
            pltpu.async_copy(tb.at[colv.at[ci + 1]], gcbuf.at[other], gsem)

        pltpu.make_async_copy(trow.at[rowv.at[ci]], grbuf.at[slot], gsem).wait()
        pltpu.make_async_copy(tb.at[colv.at[ci]], gcbuf.at[slot], gsem).wait()
        off = ebase + ci * CH
        pltpu.async_copy(grbuf.at[slot], drow.at[pl.ds(off, CH)], ssem)
        pltpu.async_copy(gcbuf.at[slot], db.at[pl.ds(off, CH)], ssem)
        return carry

    lax.fori_loop(0, NCHUNK, body, 0)
    pltpu.make_async_copy(grbuf.at[0], drow.at[pl.ds(0, CH)], ssem).wait()
    pltpu.make_async_copy(gcbuf.at[0], db.at[pl.ds(0, CH)], ssem).wait()


def _gather(trow, tb, rowm, colm):
    kern = pl.kernel(
        _gather_kernel,
        out_type=[jax.ShapeDtypeStruct((E, 128), F32),
                  jax.ShapeDtypeStruct((E, 64), F32)],
        mesh=_MESH,
        scratch_types=[pltpu.VMEM((NCHUNK, CH), jnp.int32),
                       pltpu.VMEM((NCHUNK, CH), jnp.int32),
                       pltpu.VMEM((2, CH, 128), F32),
                       pltpu.VMEM((2, CH, 64), F32),
                       pltpu.SemaphoreType.DMA,
                       pltpu.SemaphoreType.DMA],
        compiler_params=_SC_PARAMS,
    )
    return kern(trow, tb, rowm, colm)


def _scatter_kernel(m, colm, zeros, aggp, colv, mbuf, aggsh, sem):
    cid = lax.axis_index("c")
    sid = lax.axis_index("s")
    wid = sid * NC + cid

    @pl.when(sid < 10)
    def _():
        pltpu.sync_copy(zeros, aggsh.at[pl.ds(sid * 1000, 1000)])

    pltpu.sync_copy(colm.at[wid], colv)
    plsc.subcore_barrier()
    pltpu.async_copy(m.at[pl.ds(wid * EPT, CH)], mbuf.at[0], sem)

    def body(ci, carry):
        slot = lax.rem(ci, 2)

        @pl.when(ci + 1 < NCHUNK)
        def _():
            off = wid * EPT + (ci + 1) * CH
            pltpu.async_copy(m.at[pl.ds(off, CH)], mbuf.at[1 - slot], sem)

        pltpu.make_async_copy(m.at[pl.ds(0, CH)], mbuf.at[slot], sem).wait()
        pltpu.sync_copy(mbuf.at[slot], aggsh.at[colv.at[ci]], add=True)
        return carry

    lax.fori_loop(0, NCHUNK, body, 0)
    plsc.subcore_barrier()

    @pl.when(sid < 10)
    def _():
        pltpu.sync_copy(aggsh.at[pl.ds(sid * 1000, 1000)],
                        aggp.at[cid, pl.ds(sid * 1000, 1000)])


def _scatter(m, colm, zeros):
    kern = pl.kernel(
        _scatter_kernel,
        out_type=jax.ShapeDtypeStruct((NC, N, 64), F32),
        mesh=_MESH,
        scratch_types=[pltpu.VMEM((NCHUNK, CH), jnp.int32),
                       pltpu.VMEM((2, CH, 64), F32),
                       pltpu.VMEM_SHARED((N, 64), F32),
                       pltpu.SemaphoreType.DMA],
        compiler_params=_SC_PARAMS,
    )
    return kern(m, colm, zeros)


def _cnt_kernel(colm, zeros16, ones16, cntp, colv, obuf, cntsh):
    cid = lax.axis_index("c")
    sid = lax.axis_index("s")
    wid = sid * NC + cid

    @pl.when(sid < 10)
    def _():
        pltpu.sync_copy(zeros16, cntsh.at[pl.ds(sid * 1000, 1000)])

    pltpu.sync_copy(ones16, obuf)
    pltpu.sync_copy(colm.at[wid], colv)
    plsc.subcore_barrier()

    def body(ci, carry):
        pltpu.sync_copy(obuf, cntsh.at[colv.at[ci]], add=True)
        return carry

    lax.fori_loop(0, NCHUNK, body, 0)
    plsc.subcore_barrier()

    @pl.when(sid < 10)
    def _():
        pltpu.sync_copy(cntsh.at[pl.ds(sid * 1000, 1000)],
                        cntp.at[cid, pl.ds(sid * 1000, 1000)])


def _cnt(colm, zeros16, ones16):
    kern = pl.kernel(
        _cnt_kernel,
        out_type=jax.ShapeDtypeStruct((NC, N, 16), F32),
        mesh=_MESH,
        scratch_types=[pltpu.VMEM((NCHUNK, CH), jnp.int32),
                       pltpu.VMEM((CH, 16), F32),
                       pltpu.VMEM_SHARED((N, 16), F32)],
        compiler_params=_SC_PARAMS,
    )
    return kern(colm, zeros16, ones16)


def kernel(x, edge_index, edge_attr, u, batch, params):
    p = params
    row = edge_index[0]
    col = edge_index[1]
    rowm = row.reshape(NW, NCHUNK, CH)
    colm = col.reshape(NW, NCHUNK, CH)
    batch_c = batch.reshape(N, 1)
    batch_r = batch.reshape(1, N)
    zeros64 = jnp.zeros((1000, 64), F32)
    zeros16 = jnp.zeros((1000, 16), F32)
    ones16 = jnp.ones((CH, 16), F32)

    Ew1 = p['Ew1']
    Nw1a = p['Nw1a']
    Nw2a = p['Nw2a']
    Gw1 = p['Gw1']

    def r2(b):
        return b.reshape(1, -1)

    eat = edge_attr.T
    h, g = _enc_node(x, p['Wn'], r2(p['bn']), u, p['Wg'], r2(p['bg']))
    cntp = _cnt(colm, zeros16, ones16)

    node_w = (Nw2a[0:64], Nw2a[64:128], Nw2a[128:160], r2(p['Nb2a']),
              p['Nw2b'], r2(p['Nb2b']),
              Gw1[0:32], Gw1[32:96], r2(p['Gb1']), p['Gw2'], r2(p['Gb2']),
              p['Vw1'], r2(p['Vb1']), p['Vw2'], r2(p['Vb2']))

    def tables_gather(h, g):
        trow, tb = _tables(h, batch_c, g,
                           Ew1[0:64], Ew1[160:192], r2(p['Eb1']),
                           Ew1[64:128], Nw1a[0:64], r2(p['Nb1a']))
        return _gather(trow, tb, rowm, colm)

    edge_w = (Ew1[128:160], p['Ew2'], r2(p['Eb2']),
              Nw1a[64:96], p['Nw1b'], r2(p['Nb1b']))

    drow, db = tables_gather(h, g)
    e, m = _edge_stage1(drow, db, eat, p['We'], r2(p['be']), *edge_w)
    aggp = _scatter(m, colm, zeros64)
    h, g, _ = _node_stage(h, aggp, cntp, batch_c, batch_r, g, *node_w)

    drow, db = tables_gather(h, g)
    m, eoT = _edge_stage2(drow, db, e, *edge_w, p['Dw'].reshape(1, 32), r2(p['Db']))
    aggp = _scatter(m, colm, zeros64)
    h, g, value = _node_stage(h, aggp, cntp, batch_c, batch_r, g, *node_w)

    return eoT.reshape(E, 1), value
